# Initial kernel scaffold; baseline (speedup 1.0000x reference)
import jax
import jax.numpy as jnp
from jax.experimental import pallas as pl


def kernel(inp, memory_bank, w_in):
    raise NotImplementedError("write your pallas kernel here")


# speedup vs baseline: 2.2125x; 2.2125x over previous
import jax
import jax.numpy as jnp
from jax import lax
from jax.experimental import pallas as pl
from jax.experimental.pallas import tpu as pltpu


_GROUP = 8


def _pointer_kernel(q_ref, m_ref, w_ref, o_ref):
    GB, T, D = q_ref.shape
    S = m_ref.shape[1]

    w = w_ref[...].astype(jnp.bfloat16)
    q_all = q_ref[...].astype(jnp.bfloat16).reshape(GB * T, D)
    qp = lax.dot_general(q_all, w, (((1,), (1,)), ((), ())),
                         preferred_element_type=jnp.float32)
    qp = qp.astype(jnp.bfloat16).reshape(GB, T, D)

    m = m_ref[...].astype(jnp.bfloat16)

    group = []
    for b in range(GB):
        scores = lax.dot_general(qp[b], m[b], (((1,), (1,)), ((), ())),
                                 preferred_element_type=jnp.float32)
        mx = jnp.max(scores, axis=-1, keepdims=True)
        shifted = scores - mx
        lse = jnp.log(jnp.sum(jnp.exp(shifted), axis=-1, keepdims=True))
        group.append(shifted - lse)
    o_ref[...] = jnp.stack(group, axis=1)


def kernel(inp, memory_bank, w_in):
    B, T, D = inp.shape
    _, S, _ = memory_bank.shape

    gb = _GROUP if B % _GROUP == 0 else B
    grid = (B // gb,)

    return pl.pallas_call(
        _pointer_kernel,
        out_shape=jax.ShapeDtypeStruct((T, B, S), jnp.float32),
        grid=grid,
        in_specs=[
            pl.BlockSpec((gb, T, D), lambda g: (g, 0, 0)),
            pl.BlockSpec((gb, S, D), lambda g: (g, 0, 0)),
            pl.BlockSpec((D, D), lambda g: (0, 0)),
        ],
        out_specs=pl.BlockSpec((T, gb, S), lambda g: (0, g, 0)),
        compiler_params=pltpu.CompilerParams(
            dimension_semantics=("parallel",),
            vmem_limit_bytes=56 * 1024 * 1024,
        ),
    )(inp, memory_bank, w_in)
